# baseline (speedup 1.0000x reference)
import functools

import jax
import jax.numpy as jnp
from jax.experimental import pallas as pl
from jax.experimental.pallas import tpu as pltpu

_B, _S, _D, _E, _DFF, _TOPK = 1, 2048, 1024, 8, 2048, 2
_TS = 512
_NS = _S // _TS
_FC = 512


def _route_kernel(x_ref, wg_ref, logits_ref, wts_ref, idx_ref):
    x = x_ref[...]
    r = jnp.dot(x, wg_ref[...], preferred_element_type=jnp.float32)
    logits = jnp.mean(r, axis=0, keepdims=True)
    logits_ref[...] = logits
    m = jnp.max(logits)
    ex = jnp.exp(logits - m)
    p = ex / jnp.sum(ex)
    i1 = jnp.argmax(p)
    w1 = jnp.max(p)
    iota = jax.lax.broadcasted_iota(jnp.int32, (1, _E), 1)
    p2 = jnp.where(iota == i1, -jnp.inf, p)
    i2 = jnp.argmax(p2)
    w2 = jnp.max(p2)
    wts_ref[...] = jnp.concatenate(
        [w1.reshape(1, 1), w2.reshape(1, 1)], axis=1)
    idx_ref[...] = jnp.concatenate(
        [i1.astype(jnp.int32).reshape(1, 1), i2.astype(jnp.int32).reshape(1, 1)],
        axis=1)


def _ffn_kernel(idx_sm, wts_sm, x_ref, w1_ref, b1_ref, w2_ref, b2_ref,
                out_ref, w1bf_ref, w2bf_ref):
    k = pl.program_id(0)
    si = pl.program_id(1)

    @pl.when(si == 0)
    def _():
        w1bf_ref[...] = w1_ref[0].astype(jnp.bfloat16)
        w2bf_ref[...] = w2_ref[0].astype(jnp.bfloat16)

    x = x_ref[...].astype(jnp.bfloat16)
    o = None
    for c in range(0, _DFF, _FC):
        hc = jnp.dot(x, w1bf_ref[:, c:c + _FC],
                     preferred_element_type=jnp.float32)
        hc = jax.nn.gelu(hc + b1_ref[0, :, c:c + _FC])
        oc = jnp.dot(hc.astype(jnp.bfloat16), w2bf_ref[c:c + _FC, :],
                     preferred_element_type=jnp.float32)
        o = oc if o is None else o + oc
    contrib = wts_sm[k] * (o + b2_ref[0])

    @pl.when(k == 0)
    def _():
        out_ref[pl.ds(si * _TS, _TS), :] = contrib

    @pl.when(k > 0)
    def _():
        out_ref[pl.ds(si * _TS, _TS), :] = (
            out_ref[pl.ds(si * _TS, _TS), :] + contrib)


@jax.jit
def kernel(hidden_states, W_gate, W1, b1, W2, b2):
    x2 = hidden_states.reshape(_S, _D)

    logits, wts, idx = pl.pallas_call(
        _route_kernel,
        out_shape=(
            jax.ShapeDtypeStruct((1, _E), jnp.float32),
            jax.ShapeDtypeStruct((1, _TOPK), jnp.float32),
            jax.ShapeDtypeStruct((1, _TOPK), jnp.int32),
        ),
    )(x2, W_gate)

    grid_spec = pltpu.PrefetchScalarGridSpec(
        num_scalar_prefetch=2,
        grid=(_TOPK, _NS),
        in_specs=[
            pl.BlockSpec((_TS, _D), lambda k, si, idx_s, wts_s: (si, 0)),
            pl.BlockSpec((1, _D, _DFF),
                         lambda k, si, idx_s, wts_s: (idx_s[k], 0, 0)),
            pl.BlockSpec((1, 1, _DFF),
                         lambda k, si, idx_s, wts_s: (idx_s[k], 0, 0)),
            pl.BlockSpec((1, _DFF, _D),
                         lambda k, si, idx_s, wts_s: (idx_s[k], 0, 0)),
            pl.BlockSpec((1, 1, _D),
                         lambda k, si, idx_s, wts_s: (idx_s[k], 0, 0)),
        ],
        out_specs=pl.BlockSpec((_S, _D), lambda k, si, idx_s, wts_s: (0, 0)),
        scratch_shapes=[
            pltpu.VMEM((_D, _DFF), jnp.bfloat16),
            pltpu.VMEM((_DFF, _D), jnp.bfloat16),
        ],
    )
    out = pl.pallas_call(
        _ffn_kernel,
        grid_spec=grid_spec,
        out_shape=jax.ShapeDtypeStruct((_S, _D), jnp.float32),
        compiler_params=pltpu.CompilerParams(
            dimension_semantics=("arbitrary", "arbitrary")),
    )(idx.reshape(_TOPK), wts.reshape(_TOPK), x2, W1,
      b1.reshape(_E, 1, _DFF), W2, b2.reshape(_E, 1, _D))

    return (out.reshape(_B, _S, _D), logits)


# speedup vs baseline: 1.1379x; 1.1379x over previous
import functools

import jax
import jax.numpy as jnp
from jax.experimental import pallas as pl
from jax.experimental.pallas import tpu as pltpu

_B, _S, _D, _E, _DFF, _TOPK = 1, 2048, 1024, 8, 2048, 2
_TS = 256
_NS = _S // _TS


def _moe_kernel(x_ref, wg_ref, b1_ref, b2_ref, w1_hbm, w2_hbm,
                out_ref, logits_ref,
                w1v_ref, w2v_ref, idx_sm, wts_sm, sem1, sem2):
    k = pl.program_id(0)

    @pl.when(k == 0)
    def _():
        xbar = jnp.mean(x_ref[...], axis=0, keepdims=True)
        logits = jnp.dot(xbar, wg_ref[...],
                         preferred_element_type=jnp.float32)
        logits_ref[...] = logits
        m = jnp.max(logits)
        ex = jnp.exp(logits - m)
        p = ex / jnp.sum(ex)
        i1 = jnp.argmax(p).astype(jnp.int32)
        v1 = jnp.max(p)
        iota = jax.lax.broadcasted_iota(jnp.int32, (1, _E), 1)
        p2 = jnp.where(iota == i1, -jnp.inf, p)
        i2 = jnp.argmax(p2).astype(jnp.int32)
        v2 = jnp.max(p2)
        idx_sm[0] = i1
        idx_sm[1] = i2
        wts_sm[0] = v1
        wts_sm[1] = v2
        pltpu.make_async_copy(w1_hbm.at[i1], w1v_ref.at[0], sem1.at[0]).start()
        pltpu.make_async_copy(w2_hbm.at[i1], w2v_ref.at[0], sem2.at[0]).start()
        pltpu.make_async_copy(w1_hbm.at[i2], w1v_ref.at[1], sem1.at[1]).start()
        pltpu.make_async_copy(w2_hbm.at[i2], w2v_ref.at[1], sem2.at[1]).start()

    e = idx_sm[k]
    wgt = wts_sm[k]
    pltpu.make_async_copy(w1_hbm.at[e], w1v_ref.at[k], sem1.at[k]).wait()
    pltpu.make_async_copy(w2_hbm.at[e], w2v_ref.at[k], sem2.at[k]).wait()
    w1bf = w1v_ref[k].astype(jnp.bfloat16)
    w2bf = w2v_ref[k].astype(jnp.bfloat16)
    b1e = b1_ref[e]
    b2e = b2_ref[e]
    for si in range(_NS):
        xt = x_ref[pl.ds(si * _TS, _TS), :].astype(jnp.bfloat16)
        h = jnp.dot(xt, w1bf, preferred_element_type=jnp.float32)
        h = jax.nn.gelu(h + b1e)
        o = jnp.dot(h.astype(jnp.bfloat16), w2bf,
                    preferred_element_type=jnp.float32)
        contrib = wgt * (o + b2e)

        @pl.when(k == 0)
        def _():
            out_ref[pl.ds(si * _TS, _TS), :] = contrib

        @pl.when(k > 0)
        def _():
            out_ref[pl.ds(si * _TS, _TS), :] = (
                out_ref[pl.ds(si * _TS, _TS), :] + contrib)


@jax.jit
def kernel(hidden_states, W_gate, W1, b1, W2, b2):
    x2 = hidden_states.reshape(_S, _D)

    out, logits = pl.pallas_call(
        _moe_kernel,
        grid=(_TOPK,),
        in_specs=[
            pl.BlockSpec((_S, _D), lambda k: (0, 0)),
            pl.BlockSpec((_D, _E), lambda k: (0, 0)),
            pl.BlockSpec((_E, 1, _DFF), lambda k: (0, 0, 0)),
            pl.BlockSpec((_E, 1, _D), lambda k: (0, 0, 0)),
            pl.BlockSpec(memory_space=pl.ANY),
            pl.BlockSpec(memory_space=pl.ANY),
        ],
        out_specs=(
            pl.BlockSpec((_S, _D), lambda k: (0, 0)),
            pl.BlockSpec((1, _E), lambda k: (0, 0)),
        ),
        out_shape=(
            jax.ShapeDtypeStruct((_S, _D), jnp.float32),
            jax.ShapeDtypeStruct((1, _E), jnp.float32),
        ),
        scratch_shapes=[
            pltpu.VMEM((_TOPK, _D, _DFF), jnp.float32),
            pltpu.VMEM((_TOPK, _DFF, _D), jnp.float32),
            pltpu.SMEM((_TOPK,), jnp.int32),
            pltpu.SMEM((_TOPK,), jnp.float32),
            pltpu.SemaphoreType.DMA((_TOPK,)),
            pltpu.SemaphoreType.DMA((_TOPK,)),
        ],
        compiler_params=pltpu.CompilerParams(
            dimension_semantics=("arbitrary",)),
    )(x2, W_gate, b1.reshape(_E, 1, _DFF), b2.reshape(_E, 1, _D), W1, W2)

    return (out.reshape(_B, _S, _D), logits)
